# baseline (speedup 1.0000x reference)
import functools

import jax
import jax.numpy as jnp
from jax import lax
from jax.experimental import pallas as pl
from jax.experimental.pallas import tpu as pltpu
from jax.experimental.pallas import tpu_sc as plsc

BATCH = 4
SEQ_LEN = 4096
D_MODEL = 2048
NUM_BUCKETS = 64
MAX_SKIP_LAYERS = 12
_MAX_SKIPPED_FLOPS = float(MAX_SKIP_LAYERS * 12 * D_MODEL * D_MODEL * SEQ_LEN)

_N = BATCH * SEQ_LEN
_NC = 2
_NS = 16
_NW = _NC * _NS
_TPW = _N // _NW
_C = 16
_NCHUNK = _TPW // _C


def _sc_body(x_hbm, csf_hbm, tab_hbm, out_hbm, csf_v, idx_v, x_v, emb_v, sem):
    wid = lax.axis_index("s") * _NC + lax.axis_index("c")
    base = wid * _TPW

    def chunk(ci, carry):
        t0 = base + ci * _C
        pltpu.sync_copy(csf_hbm.at[pl.ds(t0, _C)], csf_v)
        frac = csf_v[...] / jnp.float32(_MAX_SKIPPED_FLOPS)
        idx = (frac * jnp.float32(NUM_BUCKETS)).astype(jnp.int32)
        idx_v[...] = jnp.clip(idx, 0, NUM_BUCKETS - 1)
        gather = pltpu.async_copy(tab_hbm.at[idx_v], emb_v, sem)
        pltpu.sync_copy(x_hbm.at[pl.ds(t0, _C)], x_v)
        gather.wait()

        def tok(t, c2):
            for j in range(D_MODEL // 16):
                s = pl.ds(j * 16, 16)
                x_v[t, s] = x_v[t, s] + emb_v[t, s]
            return c2

        lax.fori_loop(0, _C, tok, 0)
        pltpu.sync_copy(x_v, out_hbm.at[pl.ds(t0, _C)])
        return carry

    lax.fori_loop(0, _NCHUNK, chunk, 0)


@functools.partial(jax.jit, donate_argnums=())
def _sc_call(x2, csf1, table):
    mesh = plsc.VectorSubcoreMesh(core_axis_name="c", subcore_axis_name="s")
    f = functools.partial(
        pl.kernel,
        out_type=jax.ShapeDtypeStruct((_N, D_MODEL), jnp.float32),
        mesh=mesh,
        scratch_types=[
            pltpu.VMEM((_C,), jnp.float32),
            pltpu.VMEM((_C,), jnp.int32),
            pltpu.VMEM((_C, D_MODEL), jnp.float32),
            pltpu.VMEM((_C, D_MODEL), jnp.float32),
            pltpu.SemaphoreType.DMA,
        ],
    )(_sc_body)
    return f(x2, csf1, table)


def kernel(x, cumulative_skipped_flops, step_embeddings_weight):
    x2 = x.reshape(_N, D_MODEL)
    csf1 = cumulative_skipped_flops.reshape(_N)
    out = _sc_call(x2, csf1, step_embeddings_weight)
    return out.reshape(BATCH, SEQ_LEN, D_MODEL)
